# Initial kernel scaffold; baseline (speedup 1.0000x reference)
import jax
import jax.numpy as jnp
from jax.experimental import pallas as pl


def kernel(inputs):
    raise NotImplementedError("write your pallas kernel here")


# speedup vs baseline: 6.3001x; 6.3001x over previous
import functools
import math

import jax
import jax.numpy as jnp
from jax.experimental import pallas as pl

_NUM_UNITS = 1024
_SCALE = math.sqrt(float(_NUM_UNITS))
_NEG2LN1E4 = -2.0 * math.log(10000.0) / float(_NUM_UNITS)
_HALF_PI = math.pi / 2.0


def _pe_tile_kernel(o_ref, *, tile_t):
    pid = pl.program_id(0)
    t0 = (pid * tile_t).astype(jnp.float32)
    irow = jax.lax.broadcasted_iota(jnp.int32, (tile_t, _NUM_UNITS), 0)
    rows = irow.astype(jnp.float32) + t0
    icol = jax.lax.broadcasted_iota(jnp.int32, (tile_t, _NUM_UNITS), 1)
    fcol = icol.astype(jnp.float32)
    inv_freq = jnp.exp(fcol * _NEG2LN1E4)
    phase = (icol & 1).astype(jnp.float32) * _HALF_PI
    val = jnp.sin(rows * inv_freq + phase) * _SCALE
    val = jnp.where(rows == 0.0, 0.0, val)
    o_ref[...] = jnp.broadcast_to(val[None], o_ref.shape)


def kernel(inputs):
    n, t = inputs.shape
    tile_t = 256
    out = pl.pallas_call(
        functools.partial(_pe_tile_kernel, tile_t=tile_t),
        grid=(t // tile_t,),
        out_specs=pl.BlockSpec((n, tile_t, _NUM_UNITS), lambda i: (0, i, 0)),
        out_shape=jax.ShapeDtypeStruct((n, t, _NUM_UNITS), jnp.float32),
    )()
    return out


# speedup vs baseline: 14.5167x; 2.3042x over previous
import functools
import math

import jax
import jax.numpy as jnp
from jax.experimental import pallas as pl
import jax.experimental.pallas.tpu as pltpu

_NUM_UNITS = 1024
_SCALE = math.sqrt(float(_NUM_UNITS))
_NEG2LN1E4 = -2.0 * math.log(10000.0) / float(_NUM_UNITS)


def _pe_tile_kernel(o_ref, s_ref, c_ref, *, tile_t):
    pid = pl.program_id(0)
    col = jax.lax.broadcasted_iota(jnp.int32, (1, _NUM_UNITS), 1)
    w = jnp.exp(col.astype(jnp.float32) * _NEG2LN1E4)

    @pl.when(pid == 0)
    def _build_lo_tables():
        t_lo = jax.lax.broadcasted_iota(jnp.int32, (tile_t, _NUM_UNITS), 0)
        a_lo = t_lo.astype(jnp.float32) * w
        s_ref[...] = jnp.sin(a_lo)
        c_ref[...] = jnp.cos(a_lo)

    a_hi = (pid * tile_t).astype(jnp.float32) * w
    sh = jnp.sin(a_hi)
    ch = jnp.cos(a_hi)
    even = (col & 1) == 0
    p = jnp.where(even, sh, ch) * _SCALE
    q = jnp.where(even, ch, -sh) * _SCALE
    val = p * c_ref[...] + q * s_ref[...]
    o_ref[...] = jnp.broadcast_to(val[None], o_ref.shape)

    @pl.when(pid == 0)
    def _zero_row0():
        o_ref[:, 0:1, :] = jnp.zeros_like(o_ref[:, 0:1, :])


def kernel(inputs):
    n, t = inputs.shape
    tile_t = 256
    out = pl.pallas_call(
        functools.partial(_pe_tile_kernel, tile_t=tile_t),
        grid=(t // tile_t,),
        out_specs=pl.BlockSpec((n, tile_t, _NUM_UNITS), lambda i: (0, i, 0)),
        out_shape=jax.ShapeDtypeStruct((n, t, _NUM_UNITS), jnp.float32),
        scratch_shapes=[
            pltpu.VMEM((tile_t, _NUM_UNITS), jnp.float32),
            pltpu.VMEM((tile_t, _NUM_UNITS), jnp.float32),
        ],
    )()
    return out
